# baseline (speedup 1.0000x reference)
import functools

import jax
import jax.numpy as jnp
from jax import lax
from jax.experimental import pallas as pl
from jax.experimental.pallas import tpu as pltpu
from jax.experimental.pallas import tpu_sc as plsc

_NUM_CORES = 2
_NUM_SUBCORES = 16
_NW = _NUM_CORES * _NUM_SUBCORES
_LANES = 16


def _detile_kernel(n_emb, d):
    n_full = n_emb // 128
    n_pad_rows = n_full * d + d

    mesh = plsc.VectorSubcoreMesh(
        core_axis_name="c", subcore_axis_name="s", num_cores=_NUM_CORES
    )

    ch = 4
    n_ch = 62
    nbuf = 2

    @functools.partial(
        pl.kernel,
        mesh=mesh,
        compiler_params=pltpu.CompilerParams(needs_layout_passes=False),
        out_type=jax.ShapeDtypeStruct((n_pad_rows, 128), jnp.float32),
        scratch_types=(
            [pltpu.VMEM((d, ch * 128), jnp.float32)] * 2
            + [pltpu.VMEM((ch * d, 128), jnp.float32)] * 2
            + [pltpu.SemaphoreType.DMA] * 4
        ),
    )
    def _detile(tt_hbm, tail_hbm, lin_hbm, ti0, ti1,
                to0, to1,
                sem_i0, sem_i1,
                sem_o0, sem_o1):
        cid = lax.axis_index("c")
        sid = lax.axis_index("s")
        wid = sid * _NUM_CORES + cid
        tin = (ti0, ti1)
        tout = (to0, to1)
        sem_i = (sem_i0, sem_i1)
        sem_o = (sem_o0, sem_o1)

        lane = lax.iota(jnp.int32, _LANES)
        row_c = [jnp.full((_LANES,), r, jnp.int32) for r in range(ch * d)]
        f_r = [lax.rem(lane + r, jnp.int32(d)) for r in range(d)]
        col_r = [(lane * d + lax.rem(lane + r, jnp.int32(d))) for r in range(d)]
        c_16 = [lane + i * _LANES for i in range(ch * d)]

        n_rem = n_full % _NW
        n_base = n_full // _NW
        start_w = wid * n_base + lax.min(wid, jnp.int32(n_rem))
        n_w = jnp.int32(n_base) + jnp.where(wid < n_rem, 1, 0).astype(jnp.int32)

        def _chunk_c(m):
            return start_w + lax.min(m * ch, n_w - ch)

        def _start_in(m, b):
            c = _chunk_c(m)
            return pltpu.async_copy(
                tt_hbm.at[:, pl.ds(c * 128, ch * 128)], tin[b], sem_i[b]
            )

        for b in range(nbuf - 1):
            _start_in(jnp.int32(b), b)
        for b in range(nbuf):
            pltpu.async_copy(
                lin_hbm.at[pl.ds(0, ch * d)], tout[b], sem_o[b]
            )

        def _tile(m, b):
            pltpu.make_async_copy(
                tt_hbm.at[:, pl.ds(0, ch * 128)], tin[b], sem_i[b]
            ).wait()
            _start_in(m + nbuf - 1, (b + nbuf - 1) % nbuf)
            pltpu.make_async_copy(
                lin_hbm.at[pl.ds(0, ch * d)], tout[b], sem_o[b]
            ).wait()
            for i in range(ch * d):
                for r in range(d):
                    v = plsc.load_gather(tin[b], [f_r[r], c_16[i]])
                    plsc.store_scatter(tout[b], [row_c[i], col_r[r]], v)
            c = _chunk_c(m)
            pltpu.async_copy(
                tout[b], lin_hbm.at[pl.ds(c * d, ch * d)], sem_o[b]
            )

        def _quad(kk, carry):
            for b in range(nbuf):
                _tile(kk * nbuf + b, b)
            return carry

        lax.fori_loop(0, n_ch // nbuf, _quad, 0)
        for b in range(nbuf - 1):
            pltpu.make_async_copy(
                tt_hbm.at[:, pl.ds(0, ch * 128)], tin[b], sem_i[b]
            ).wait()
        for b in range(nbuf):
            pltpu.make_async_copy(
                lin_hbm.at[pl.ds(0, ch * d)], tout[b], sem_o[b]
            ).wait()

        @pl.when(sid == 0)
        def _():
            pltpu.async_copy(
                tail_hbm, lin_hbm.at[pl.ds(n_full * d, d)], sem_o0
            ).wait()

    return _detile


def _gather_kernel(rows, cols, n_rows, d):
    ipw = rows // _NW
    tpw = ipw // 128

    mesh = plsc.VectorSubcoreMesh(
        core_axis_name="c", subcore_axis_name="s", num_cores=_NUM_CORES
    )

    @functools.partial(
        pl.kernel,
        mesh=mesh,
        compiler_params=pltpu.CompilerParams(
            use_tc_tiling_on_sc=False, needs_layout_passes=False
        ),
        out_type=jax.ShapeDtypeStruct((cols, rows // 128, d, 128), jnp.float32),
        scratch_types=[
            pltpu.VMEM((cols * ipw,), jnp.int32),
            pltpu.VMEM((ipw, d), jnp.float32),
            pltpu.VMEM((ipw, d), jnp.float32),
            pltpu.VMEM((tpw, d, 128), jnp.float32),
            pltpu.VMEM((tpw, d, 128), jnp.float32),
            pltpu.SemaphoreType.DMA,
            pltpu.SemaphoreType.DMA,
            pltpu.SemaphoreType.DMA,
            pltpu.SemaphoreType.DMA,
            pltpu.SemaphoreType.DMA,
        ],
    )
    def _embed(xt_hbm, table_hbm, out_hbm, idx_v, rows_v0, rows_v1,
               out_v0, out_v1, sem_x, sem_g0, sem_g1, sem_o0, sem_o1):
        wid = lax.axis_index("s") * _NUM_CORES + lax.axis_index("c")
        base = wid * ipw
        rows_bufs = (rows_v0, rows_v1)
        out_bufs = (out_v0, out_v1)
        sem_g = (sem_g0, sem_g1)
        sem_o = (sem_o0, sem_o1)

        idx_copies = [
            pltpu.async_copy(
                xt_hbm.at[j, pl.ds(base, ipw)],
                idx_v.at[pl.ds(j * ipw, ipw)],
                sem_x,
            )
            for j in range(cols)
        ]
        for c in idx_copies:
            c.wait()

        lane = lax.iota(jnp.int32, _LANES)
        col_of = [jnp.full((_LANES,), f, jnp.int32) for f in range(d)]

        def _start_gather(j, b):
            return pltpu.async_copy(
                table_hbm.at[idx_v.at[pl.ds(j * ipw, ipw)]],
                rows_bufs[b],
                sem_g[b],
            )

        def _transpose(rows_v, out_v):
            for t in range(tpw):
                def _body(g, carry, t=t):
                    r16 = t * 128 + g * _LANES + lane
                    for f in range(d):
                        v = plsc.load_gather(rows_v, [r16, col_of[f]])
                        out_v[t, f, pl.ds(g * _LANES, _LANES)] = v
                    return carry
                lax.fori_loop(0, 128 // _LANES, _body, 0)

        _start_gather(0, 0)
        for b in range(2):
            pltpu.async_copy(
                out_hbm.at[b, pl.ds(wid * tpw, tpw)], out_bufs[b], sem_o[b]
            )

        def _chunk(j2, carry):
            for b in range(2):
                j = j2 * 2 + b
                pltpu.make_async_copy(
                    table_hbm.at[idx_v.at[pl.ds(0, ipw)]],
                    rows_bufs[b],
                    sem_g[b],
                ).wait()
                jn = lax.rem(j + 1, cols)
                _start_gather(jn, (b + 1) % 2)
                pltpu.make_async_copy(
                    out_hbm.at[0, pl.ds(wid * tpw, tpw)], out_bufs[b], sem_o[b]
                ).wait()
                _transpose(rows_bufs[b], out_bufs[b])
                pltpu.async_copy(
                    out_bufs[b], out_hbm.at[j, pl.ds(wid * tpw, tpw)], sem_o[b]
                )
            return carry

        lax.fori_loop(0, cols // 2, _chunk, 0)
        pltpu.make_async_copy(
            table_hbm.at[idx_v.at[pl.ds(0, ipw)]], rows_bufs[0], sem_g[0]
        ).wait()
        for b in range(2):
            pltpu.make_async_copy(
                out_hbm.at[0, pl.ds(wid * tpw, tpw)], out_bufs[b], sem_o[b]
            ).wait()

    return _embed


def kernel(x, table):
    rows, cols = x.shape
    n_emb, d = table.shape
    n_full = n_emb // 128
    n_pad = n_full * 128 + 128

    xt = x.T
    tt = table.T
    tail = jnp.concatenate(
        [table[n_full * 128:, :],
         jnp.zeros((n_pad - n_emb, d), jnp.float32)]
    ).reshape(d, 128)

    lin = _detile_kernel(n_emb, d)(tt, tail)
    out4 = _gather_kernel(rows, cols, n_pad, d)(
        xt, lin.reshape(n_pad, d)
    )
    return out4.transpose(1, 3, 0, 2).reshape(rows, cols, d)


# speedup vs baseline: 1.6036x; 1.6036x over previous
import functools

import jax
import jax.numpy as jnp
from jax import lax
from jax.experimental import pallas as pl
from jax.experimental.pallas import tpu as pltpu
from jax.experimental.pallas import tpu_sc as plsc

_NUM_CORES = 2
_NUM_SUBCORES = 16
_NW = _NUM_CORES * _NUM_SUBCORES
_LANES = 16


def _detile_kernel(n_emb, d):
    n_full = n_emb // 128
    n_pad_rows = n_full * d + d

    mesh = plsc.VectorSubcoreMesh(
        core_axis_name="c", subcore_axis_name="s", num_cores=_NUM_CORES
    )

    ch = 4
    n_ch = 62
    nbuf = 2

    @functools.partial(
        pl.kernel,
        mesh=mesh,
        compiler_params=pltpu.CompilerParams(needs_layout_passes=False),
        out_type=jax.ShapeDtypeStruct((n_pad_rows, 128), jnp.float32),
        scratch_types=(
            [pltpu.VMEM((d, ch * 128), jnp.float32)] * 2
            + [pltpu.VMEM((ch * d, 128), jnp.float32)] * 2
            + [pltpu.SemaphoreType.DMA] * 4
        ),
    )
    def _detile(tt_hbm, tail_hbm, lin_hbm, ti0, ti1,
                to0, to1,
                sem_i0, sem_i1,
                sem_o0, sem_o1):
        cid = lax.axis_index("c")
        sid = lax.axis_index("s")
        wid = sid * _NUM_CORES + cid
        tin = (ti0, ti1)
        tout = (to0, to1)
        sem_i = (sem_i0, sem_i1)
        sem_o = (sem_o0, sem_o1)

        lane = lax.iota(jnp.int32, _LANES)
        col_f = [(lane * d + f).astype(jnp.int32) for f in range(d)]
        row_c = [jnp.full((_LANES,), r, jnp.int32) for r in range(ch * d)]

        n_rem = n_full % _NW
        n_base = n_full // _NW
        start_w = wid * n_base + lax.min(wid, jnp.int32(n_rem))
        n_w = jnp.int32(n_base) + jnp.where(wid < n_rem, 1, 0).astype(jnp.int32)

        def _chunk_c(m):
            return start_w + lax.min(m * ch, n_w - ch)

        def _start_in(m, b):
            c = _chunk_c(m)
            return pltpu.async_copy(
                tt_hbm.at[:, pl.ds(c * 128, ch * 128)], tin[b], sem_i[b]
            )

        for b in range(nbuf - 1):
            _start_in(jnp.int32(b), b)
        for b in range(nbuf):
            pltpu.async_copy(
                lin_hbm.at[pl.ds(0, ch * d)], tout[b], sem_o[b]
            )

        def _tile(m, b):
            pltpu.make_async_copy(
                tt_hbm.at[:, pl.ds(0, ch * 128)], tin[b], sem_i[b]
            ).wait()
            _start_in(m + nbuf - 1, (b + nbuf - 1) % nbuf)
            pltpu.make_async_copy(
                lin_hbm.at[pl.ds(0, ch * d)], tout[b], sem_o[b]
            ).wait()
            for t in range(ch):
                for g in range(d * 128 // _LANES // d):
                    for f in range(d):
                        v = tin[b][f, pl.ds(t * 128 + g * _LANES, _LANES)]
                        plsc.store_scatter(
                            tout[b], [row_c[t * d + g], col_f[f]], v
                        )
            c = _chunk_c(m)
            pltpu.async_copy(
                tout[b], lin_hbm.at[pl.ds(c * d, ch * d)], sem_o[b]
            )

        def _quad(kk, carry):
            for b in range(nbuf):
                _tile(kk * nbuf + b, b)
            return carry

        lax.fori_loop(0, n_ch // nbuf, _quad, 0)
        for b in range(nbuf - 1):
            pltpu.make_async_copy(
                tt_hbm.at[:, pl.ds(0, ch * 128)], tin[b], sem_i[b]
            ).wait()
        for b in range(nbuf):
            pltpu.make_async_copy(
                lin_hbm.at[pl.ds(0, ch * d)], tout[b], sem_o[b]
            ).wait()

        @pl.when(sid == 0)
        def _():
            pltpu.async_copy(
                tail_hbm, lin_hbm.at[pl.ds(n_full * d, d)], sem_o0
            ).wait()

    return _detile


def _gather_kernel(rows, cols, n_rows, d):
    ipw = rows // _NW
    tpw = ipw // 128

    mesh = plsc.VectorSubcoreMesh(
        core_axis_name="c", subcore_axis_name="s", num_cores=_NUM_CORES
    )

    @functools.partial(
        pl.kernel,
        mesh=mesh,
        compiler_params=pltpu.CompilerParams(
            use_tc_tiling_on_sc=False, needs_layout_passes=False
        ),
        out_type=jax.ShapeDtypeStruct((cols, rows // 128, d, 128), jnp.float32),
        scratch_types=[
            pltpu.VMEM((cols * ipw,), jnp.int32),
            pltpu.VMEM((ipw, d), jnp.float32),
            pltpu.VMEM((ipw, d), jnp.float32),
            pltpu.VMEM((tpw, d, 128), jnp.float32),
            pltpu.VMEM((tpw, d, 128), jnp.float32),
            pltpu.SemaphoreType.DMA,
            pltpu.SemaphoreType.DMA,
            pltpu.SemaphoreType.DMA,
            pltpu.SemaphoreType.DMA,
            pltpu.SemaphoreType.DMA,
        ],
    )
    def _embed(xt_hbm, table_hbm, out_hbm, idx_v, rows_v0, rows_v1,
               out_v0, out_v1, sem_x, sem_g0, sem_g1, sem_o0, sem_o1):
        wid = lax.axis_index("s") * _NUM_CORES + lax.axis_index("c")
        base = wid * ipw
        rows_bufs = (rows_v0, rows_v1)
        out_bufs = (out_v0, out_v1)
        sem_g = (sem_g0, sem_g1)
        sem_o = (sem_o0, sem_o1)

        idx_copies = [
            pltpu.async_copy(
                xt_hbm.at[j, pl.ds(base, ipw)],
                idx_v.at[pl.ds(j * ipw, ipw)],
                sem_x,
            )
            for j in range(cols)
        ]
        for c in idx_copies:
            c.wait()

        lane = lax.iota(jnp.int32, _LANES)
        col_of = [jnp.full((_LANES,), f, jnp.int32) for f in range(d)]

        def _start_gather(j, b):
            return pltpu.async_copy(
                table_hbm.at[idx_v.at[pl.ds(j * ipw, ipw)]],
                rows_bufs[b],
                sem_g[b],
            )

        def _transpose(rows_v, out_v):
            for t in range(tpw):
                def _body(g, carry, t=t):
                    r16 = t * 128 + g * _LANES + lane
                    for f in range(d):
                        v = plsc.load_gather(rows_v, [r16, col_of[f]])
                        out_v[t, f, pl.ds(g * _LANES, _LANES)] = v
                    return carry
                lax.fori_loop(0, 128 // _LANES, _body, 0)

        _start_gather(0, 0)
        for b in range(2):
            pltpu.async_copy(
                out_hbm.at[b, pl.ds(wid * tpw, tpw)], out_bufs[b], sem_o[b]
            )

        def _chunk(j2, carry):
            for b in range(2):
                j = j2 * 2 + b
                pltpu.make_async_copy(
                    table_hbm.at[idx_v.at[pl.ds(0, ipw)]],
                    rows_bufs[b],
                    sem_g[b],
                ).wait()
                jn = lax.rem(j + 1, cols)
                _start_gather(jn, (b + 1) % 2)
                pltpu.make_async_copy(
                    out_hbm.at[0, pl.ds(wid * tpw, tpw)], out_bufs[b], sem_o[b]
                ).wait()
                _transpose(rows_bufs[b], out_bufs[b])
                pltpu.async_copy(
                    out_bufs[b], out_hbm.at[j, pl.ds(wid * tpw, tpw)], sem_o[b]
                )
            return carry

        lax.fori_loop(0, cols // 2, _chunk, 0)
        pltpu.make_async_copy(
            table_hbm.at[idx_v.at[pl.ds(0, ipw)]], rows_bufs[0], sem_g[0]
        ).wait()
        for b in range(2):
            pltpu.make_async_copy(
                out_hbm.at[0, pl.ds(wid * tpw, tpw)], out_bufs[b], sem_o[b]
            ).wait()

    return _embed


def kernel(x, table):
    rows, cols = x.shape
    n_emb, d = table.shape
    n_full = n_emb // 128
    n_pad = n_full * 128 + 128

    xt = x.T
    tt = table.T
    tail = jnp.concatenate(
        [table[n_full * 128:, :],
         jnp.zeros((n_pad - n_emb, d), jnp.float32)]
    ).reshape(d, 128)

    lin = _detile_kernel(n_emb, d)(tt, tail)
    out4 = _gather_kernel(rows, cols, n_pad, d)(
        xt, lin.reshape(n_pad, d)
    )
    return out4.transpose(1, 3, 0, 2).reshape(rows, cols, d)
